# baseline (device time: 37060 ns/iter reference)
import jax
import jax.numpy as jnp
from jax import lax
from jax.experimental import pallas as pl
from jax.experimental.pallas import tpu as pltpu

N_DEV = 16


def kernel(x, Wq, K_ext, V_ext, Wo):
    B, Sq, D = x.shape
    _, Skv, Hl, Dh = K_ext.shape
    Dq = Hl * Dh
    Dout = Wo.shape[1]
    CH = B * Sq // N_DEV
    CPB = N_DEV // B

    def body(x_ref, wq_ref, k_ref, v_ref, wo_ref, out_ref,
             acc_ref, sbuf_ref, rsbuf_ref, redb_ref, agbuf_ref,
             rs_send_sems, rs_recv_sems, ag_send_sems, ag_recv_sems):
        my = lax.axis_index("i")

        barrier_sem = pltpu.get_barrier_semaphore()
        for o in range(1, N_DEV):
            pl.semaphore_signal(
                barrier_sem, inc=1,
                device_id=(my ^ o,),
                device_id_type=pl.DeviceIdType.MESH,
            )

        wq = wq_ref[:, pl.ds(my * Dq, Dq)].astype(jnp.bfloat16)
        wo = wo_ref[pl.ds(my * Dq, Dq), :].astype(jnp.bfloat16)

        qi = lax.broadcasted_iota(jnp.int32, (Sq, Skv), 0) // 64
        ki = lax.broadcasted_iota(jnp.int32, (Sq, Skv), 1) // 64
        mask = (qi == ki) | (ki == 0) | ((qi + ki) % 3 == 0)

        def rs_send(p):
            o = my ^ p
            rdma = pltpu.make_async_remote_copy(
                src_ref=sbuf_ref.at[p],
                dst_ref=rsbuf_ref.at[o],
                send_sem=rs_send_sems.at[o],
                recv_sem=rs_recv_sems.at[o],
                device_id=(p,),
                device_id_type=pl.DeviceIdType.MESH,
            )

            @pl.when(p != my)
            def _():
                rdma.start()

        for b in range(B):
            xb = x_ref[b].astype(jnp.bfloat16)
            q = jnp.dot(xb, wq, preferred_element_type=jnp.float32)
            heads = []
            for h in range(Hl):
                qh = q[:, h * Dh:(h + 1) * Dh].astype(jnp.bfloat16)
                kh = k_ref[b, :, h, :].astype(jnp.bfloat16)
                s = lax.dot_general(
                    qh, kh, (((1,), (1,)), ((), ())),
                    preferred_element_type=jnp.float32,
                ) * 0.125
                s = jnp.where(mask, s, -1e9)
                m = jnp.max(s, axis=1, keepdims=True)
                w = jnp.exp(s - m)
                w = w / jnp.sum(w, axis=1, keepdims=True)
                vh = v_ref[b, :, h, :].astype(jnp.bfloat16)
                heads.append(jnp.dot(w.astype(jnp.bfloat16), vh,
                                     preferred_element_type=jnp.float32))
            ctx = jnp.concatenate(heads, axis=1)
            partial = jnp.dot(ctx.astype(jnp.bfloat16), wo,
                              preferred_element_type=jnp.float32)
            acc_ref[b * Sq:(b + 1) * Sq, :] = partial
            sbuf_ref[b * CPB:(b + 1) * CPB] = partial.astype(
                jnp.bfloat16).reshape(CPB, CH, Dout)

            if b == 0:
                pl.semaphore_wait(barrier_sem, N_DEV - 1)
            for p in range(b * CPB, (b + 1) * CPB):
                rs_send(p)

        rsw = []
        for o in range(1, N_DEV):
            rsw.append(pltpu.make_async_remote_copy(
                src_ref=sbuf_ref.at[0],
                dst_ref=rsbuf_ref.at[o],
                send_sem=rs_send_sems.at[o],
                recv_sem=rs_recv_sems.at[o],
                device_id=(0,),
                device_id_type=pl.DeviceIdType.MESH,
            ))
        for rdma in rsw:
            rdma.wait_recv()
        red = acc_ref[pl.ds(my * CH, CH), :] + jnp.sum(
            rsbuf_ref[1:N_DEV].astype(jnp.float32), axis=0)
        redb_ref[...] = red.astype(jnp.bfloat16)
        for rdma in rsw:
            rdma.wait_send()

        ag = []
        for o in range(1, N_DEV):
            rdma = pltpu.make_async_remote_copy(
                src_ref=redb_ref,
                dst_ref=agbuf_ref.at[my],
                send_sem=ag_send_sems.at[o],
                recv_sem=ag_recv_sems.at[o],
                device_id=(my ^ o,),
                device_id_type=pl.DeviceIdType.MESH,
            )
            rdma.start()
            ag.append(rdma)
        agbuf_ref[my] = redb_ref[...]
        for rdma in ag:
            rdma.wait_recv()
        out_ref[...] = agbuf_ref[...].astype(jnp.float32).reshape(
            B, Sq, Dout)
        for rdma in ag:
            rdma.wait_send()

    return pl.pallas_call(
        body,
        out_shape=jax.ShapeDtypeStruct((B, Sq, Dout), jnp.float32),
        in_specs=[pl.BlockSpec(memory_space=pltpu.VMEM)] * 5,
        out_specs=pl.BlockSpec(memory_space=pltpu.VMEM),
        compiler_params=pltpu.CompilerParams(collective_id=0),
        scratch_shapes=[
            pltpu.VMEM((B * Sq, Dout), jnp.float32),
            pltpu.VMEM((N_DEV, CH, Dout), jnp.bfloat16),
            pltpu.VMEM((N_DEV, CH, Dout), jnp.bfloat16),
            pltpu.VMEM((CH, Dout), jnp.bfloat16),
            pltpu.VMEM((N_DEV, CH, Dout), jnp.bfloat16),
            pltpu.SemaphoreType.DMA((N_DEV,)),
            pltpu.SemaphoreType.DMA((N_DEV,)),
            pltpu.SemaphoreType.DMA((N_DEV,)),
            pltpu.SemaphoreType.DMA((N_DEV,)),
        ],
    )(x, Wq, K_ext, V_ext, Wo)


# device time: 34490 ns/iter; 1.0745x vs baseline; 1.0745x over previous
import jax
import jax.numpy as jnp
from jax import lax
from jax.experimental import pallas as pl
from jax.experimental.pallas import tpu as pltpu

N_DEV = 16
LOG2_N = 4


def kernel(x, Wq, K_ext, V_ext, Wo):
    B, Sq, D = x.shape
    _, Skv, Hl, Dh = K_ext.shape
    Dq = Hl * Dh
    Dout = Wo.shape[1]

    CH = B * Sq // N_DEV

    def body(x_ref, wq_ref, k_ref, v_ref, wo_ref, out_ref,
             acc_ref, sbuf_ref, rsbuf_ref, redb_ref, agbuf_ref,
             rs_send_sems, rs_recv_sems, ag_send_sems, ag_recv_sems):
        my = lax.axis_index("i")

        barrier_sem = pltpu.get_barrier_semaphore()
        for o in range(1, N_DEV):
            pl.semaphore_signal(
                barrier_sem, inc=1,
                device_id=(my ^ o,),
                device_id_type=pl.DeviceIdType.MESH,
            )

        xb = x_ref[...].reshape(B * Sq, D).astype(jnp.bfloat16)
        wq = wq_ref[:, pl.ds(my * Dq, Dq)].astype(jnp.bfloat16)
        q = jnp.dot(xb, wq, preferred_element_type=jnp.float32)

        qi = lax.broadcasted_iota(jnp.int32, (Sq, Skv), 0) // 64
        ki = lax.broadcasted_iota(jnp.int32, (Sq, Skv), 1) // 64
        mask = (qi == ki) | (ki == 0) | ((qi + ki) % 3 == 0)

        ctx_rows = []
        for b in range(B):
            heads = []
            for h in range(Hl):
                qh = q[b * Sq:(b + 1) * Sq, h * Dh:(h + 1) * Dh]
                kh = k_ref[b, :, h, :].astype(jnp.bfloat16)
                s = lax.dot_general(
                    qh.astype(jnp.bfloat16), kh,
                    (((1,), (1,)), ((), ())),
                    preferred_element_type=jnp.float32,
                ) * 0.125
                w = jnp.exp(jnp.where(mask, s, -1e9))
                inv = 1.0 / jnp.sum(w, axis=1, keepdims=True)
                vh = v_ref[b, :, h, :].astype(jnp.bfloat16)
                heads.append(inv * jnp.dot(
                    w.astype(jnp.bfloat16), vh,
                    preferred_element_type=jnp.float32))
            ctx_rows.append(jnp.concatenate(heads, axis=1))
        ctx = jnp.concatenate(ctx_rows, axis=0)

        wo = wo_ref[pl.ds(my * Dq, Dq), :].astype(jnp.bfloat16)
        acc_ref[...] = jnp.dot(ctx.astype(jnp.bfloat16), wo,
                               preferred_element_type=jnp.float32)

        sbuf_ref[...] = acc_ref[...].astype(jnp.bfloat16).reshape(
            N_DEV, CH, Dout)
        pl.semaphore_wait(barrier_sem, N_DEV - 1)

        rs = []
        for o in range(1, N_DEV):
            partner = my ^ o
            rdma = pltpu.make_async_remote_copy(
                src_ref=sbuf_ref.at[partner],
                dst_ref=rsbuf_ref.at[o],
                send_sem=rs_send_sems.at[o],
                recv_sem=rs_recv_sems.at[o],
                device_id=(partner,),
                device_id_type=pl.DeviceIdType.MESH,
            )
            rdma.start()
            rs.append(rdma)
        for rdma in rs:
            rdma.wait_recv()
        red = acc_ref[pl.ds(my * CH, CH), :] + jnp.sum(
            rsbuf_ref[1:N_DEV].astype(jnp.float32), axis=0)
        redb_ref[...] = red.astype(jnp.bfloat16)
        for rdma in rs:
            rdma.wait_send()

        ag = []
        for o in range(1, N_DEV):
            partner = my ^ o
            rdma = pltpu.make_async_remote_copy(
                src_ref=redb_ref,
                dst_ref=agbuf_ref.at[my],
                send_sem=ag_send_sems.at[o],
                recv_sem=ag_recv_sems.at[o],
                device_id=(partner,),
                device_id_type=pl.DeviceIdType.MESH,
            )
            rdma.start()
            ag.append(rdma)
        agbuf_ref[my] = redb_ref[...]
        for rdma in ag:
            rdma.wait_recv()
        out_ref[...] = agbuf_ref[...].astype(jnp.float32).reshape(
            B, Sq, Dout)
        for rdma in ag:
            rdma.wait_send()

    return pl.pallas_call(
        body,
        out_shape=jax.ShapeDtypeStruct((B, Sq, Dout), jnp.float32),
        in_specs=[pl.BlockSpec(memory_space=pltpu.VMEM)] * 5,
        out_specs=pl.BlockSpec(memory_space=pltpu.VMEM),
        compiler_params=pltpu.CompilerParams(collective_id=0),
        scratch_shapes=[
            pltpu.VMEM((B * Sq, Dout), jnp.float32),
            pltpu.VMEM((N_DEV, B * Sq // N_DEV, Dout), jnp.bfloat16),
            pltpu.VMEM((N_DEV, B * Sq // N_DEV, Dout), jnp.bfloat16),
            pltpu.VMEM((B * Sq // N_DEV, Dout), jnp.bfloat16),
            pltpu.VMEM((N_DEV, B * Sq // N_DEV, Dout), jnp.bfloat16),
            pltpu.SemaphoreType.DMA((N_DEV,)),
            pltpu.SemaphoreType.DMA((N_DEV,)),
            pltpu.SemaphoreType.DMA((N_DEV,)),
            pltpu.SemaphoreType.DMA((N_DEV,)),
        ],
    )(x, Wq, K_ext, V_ext, Wo)


# device time: 34325 ns/iter; 1.0797x vs baseline; 1.0048x over previous
import jax
import jax.numpy as jnp
from jax import lax
from jax.experimental import pallas as pl
from jax.experimental.pallas import tpu as pltpu

N_DEV = 16
LOG2_N = 4


def kernel(x, Wq, K_ext, V_ext, Wo):
    B, Sq, D = x.shape
    _, Skv, Hl, Dh = K_ext.shape
    Dq = Hl * Dh
    Dout = Wo.shape[1]

    CH = B * Sq // N_DEV

    def body(x_ref, wq_ref, k_ref, v_ref, wo_ref, out_ref,
             acc_ref, sbuf_ref, rsbuf_ref, redb_ref, agbuf_ref,
             rs_send_sems, rs_recv_sems, ag_send_sems, ag_recv_sems):
        my = lax.axis_index("i")

        barrier_sem = pltpu.get_barrier_semaphore()
        for o in range(1, N_DEV):
            pl.semaphore_signal(
                barrier_sem, inc=1,
                device_id=(my ^ o,),
                device_id_type=pl.DeviceIdType.MESH,
            )

        xb = x_ref[...].reshape(B * Sq, D).astype(jnp.bfloat16)
        wq = wq_ref[:, pl.ds(my * Dq, Dq)].astype(jnp.bfloat16)
        q = jnp.dot(xb, wq, preferred_element_type=jnp.float32)

        qi = lax.broadcasted_iota(jnp.int32, (Sq, Skv), 0) // 64
        ki = lax.broadcasted_iota(jnp.int32, (Sq, Skv), 1) // 64
        mask = (qi == ki) | (ki == 0) | ((qi + ki) % 3 == 0)

        ctx_rows = []
        for b in range(B):
            heads = []
            for h in range(Hl):
                qh = q[b * Sq:(b + 1) * Sq, h * Dh:(h + 1) * Dh]
                kh = k_ref[b, h]
                s = lax.dot_general(
                    qh.astype(jnp.bfloat16), kh,
                    (((1,), (1,)), ((), ())),
                    preferred_element_type=jnp.float32,
                ) * 0.125
                s = jnp.where(mask, s, -1e9)
                m = jnp.max(s, axis=1, keepdims=True)
                w = jnp.exp(s - m)
                w = w / jnp.sum(w, axis=1, keepdims=True)
                vh = v_ref[b, h]
                heads.append(jnp.dot(w.astype(jnp.bfloat16), vh,
                                     preferred_element_type=jnp.float32))
            ctx_rows.append(jnp.concatenate(heads, axis=1))
        ctx = jnp.concatenate(ctx_rows, axis=0)

        wo = wo_ref[pl.ds(my * Dq, Dq), :].astype(jnp.bfloat16)
        acc_ref[...] = jnp.dot(ctx.astype(jnp.bfloat16), wo,
                               preferred_element_type=jnp.float32)

        sbuf_ref[...] = acc_ref[...].astype(jnp.bfloat16).reshape(
            N_DEV, CH, Dout)
        pl.semaphore_wait(barrier_sem, N_DEV - 1)

        rs = []
        for o in range(1, N_DEV):
            partner = my ^ o
            rdma = pltpu.make_async_remote_copy(
                src_ref=sbuf_ref.at[partner],
                dst_ref=rsbuf_ref.at[o],
                send_sem=rs_send_sems.at[o],
                recv_sem=rs_recv_sems.at[o],
                device_id=(partner,),
                device_id_type=pl.DeviceIdType.MESH,
            )
            rdma.start()
            rs.append(rdma)
        for rdma in rs:
            rdma.wait_recv()
        red = acc_ref[pl.ds(my * CH, CH), :] + jnp.sum(
            rsbuf_ref[1:N_DEV].astype(jnp.float32), axis=0)
        redb_ref[...] = red.astype(jnp.bfloat16)
        for rdma in rs:
            rdma.wait_send()

        ag = []
        for o in range(1, N_DEV):
            partner = my ^ o
            rdma = pltpu.make_async_remote_copy(
                src_ref=redb_ref,
                dst_ref=agbuf_ref.at[my],
                send_sem=ag_send_sems.at[o],
                recv_sem=ag_recv_sems.at[o],
                device_id=(partner,),
                device_id_type=pl.DeviceIdType.MESH,
            )
            rdma.start()
            ag.append(rdma)
        agbuf_ref[my] = redb_ref[...]
        for rdma in ag:
            rdma.wait_recv()
        out_ref[...] = agbuf_ref[...].astype(jnp.float32).reshape(
            B, Sq, Dout)
        for rdma in ag:
            rdma.wait_send()

    return pl.pallas_call(
        body,
        out_shape=jax.ShapeDtypeStruct((B, Sq, Dout), jnp.float32),
        in_specs=[pl.BlockSpec(memory_space=pltpu.VMEM)] * 5,
        out_specs=pl.BlockSpec(memory_space=pltpu.VMEM),
        compiler_params=pltpu.CompilerParams(collective_id=0),
        scratch_shapes=[
            pltpu.VMEM((B * Sq, Dout), jnp.float32),
            pltpu.VMEM((N_DEV, B * Sq // N_DEV, Dout), jnp.bfloat16),
            pltpu.VMEM((N_DEV, B * Sq // N_DEV, Dout), jnp.bfloat16),
            pltpu.VMEM((B * Sq // N_DEV, Dout), jnp.bfloat16),
            pltpu.VMEM((N_DEV, B * Sq // N_DEV, Dout), jnp.bfloat16),
            pltpu.SemaphoreType.DMA((N_DEV,)),
            pltpu.SemaphoreType.DMA((N_DEV,)),
            pltpu.SemaphoreType.DMA((N_DEV,)),
            pltpu.SemaphoreType.DMA((N_DEV,)),
        ],
    )(x, Wq,
      jnp.transpose(K_ext, (0, 2, 1, 3)).astype(jnp.bfloat16),
      jnp.transpose(V_ext, (0, 2, 1, 3)).astype(jnp.bfloat16),
      Wo)


# device time: 34287 ns/iter; 1.0809x vs baseline; 1.0011x over previous
import jax
import jax.numpy as jnp
from jax import lax
from jax.experimental import pallas as pl
from jax.experimental.pallas import tpu as pltpu

N_DEV = 16


def kernel(x, Wq, K_ext, V_ext, Wo):
    B, Sq, D = x.shape
    _, Skv, Hl, Dh = K_ext.shape
    Dq = Hl * Dh
    Dout = Wo.shape[1]
    CH = B * Sq // N_DEV
    SPB = Sq // CH

    def body(x_ref, wq_ref, k_ref, v_ref, wo_ref, out_ref,
             sbuf_ref, rsbuf_ref, redb_ref,
             rs_send_sems, rs_recv_sems, ag_send_sems, ag_recv_sems):
        my = lax.axis_index("i")

        barrier_sem = pltpu.get_barrier_semaphore()
        for o in range(1, N_DEV):
            pl.semaphore_signal(
                barrier_sem, inc=1,
                device_id=(my ^ o,),
                device_id_type=pl.DeviceIdType.MESH,
            )

        xb = x_ref[...].reshape(B * Sq, D).astype(jnp.bfloat16)
        wq = wq_ref[:, pl.ds(my * Dq, Dq)].astype(jnp.bfloat16)
        q = jnp.dot(xb, wq, preferred_element_type=jnp.float32)

        qi = lax.broadcasted_iota(jnp.int32, (Sq, Skv), 0) // 64
        ki = lax.broadcasted_iota(jnp.int32, (Sq, Skv), 1) // 64
        mask = (qi == ki) | (ki == 0) | ((qi + ki) % 3 == 0)

        ctx_rows = []
        for b in range(B):
            heads = []
            for h in range(Hl):
                qh = q[b * Sq:(b + 1) * Sq, h * Dh:(h + 1) * Dh]
                kh = k_ref[b, :, h, :].astype(jnp.bfloat16)
                s = lax.dot_general(
                    qh.astype(jnp.bfloat16), kh,
                    (((1,), (1,)), ((), ())),
                    preferred_element_type=jnp.float32,
                ) * 0.125
                s = jnp.where(mask, s, -1e9)
                m = jnp.max(s, axis=1, keepdims=True)
                w = jnp.exp(s - m)
                w = w / jnp.sum(w, axis=1, keepdims=True)
                vh = v_ref[b, :, h, :].astype(jnp.bfloat16)
                heads.append(jnp.dot(w.astype(jnp.bfloat16), vh,
                                     preferred_element_type=jnp.float32))
            ctx_rows.append(jnp.concatenate(heads, axis=1))
        ctx = jnp.concatenate(ctx_rows, axis=0)

        wo = wo_ref[pl.ds(my * Dq, Dq), :].astype(jnp.bfloat16)
        sbuf_ref[...] = jnp.dot(
            ctx.astype(jnp.bfloat16), wo,
            preferred_element_type=jnp.float32,
        ).astype(jnp.bfloat16).reshape(N_DEV, CH, Dout)
        pl.semaphore_wait(barrier_sem, N_DEV - 1)

        rs = []
        for o in range(1, N_DEV):
            partner = my ^ o
            rdma = pltpu.make_async_remote_copy(
                src_ref=sbuf_ref.at[partner],
                dst_ref=rsbuf_ref.at[o],
                send_sem=rs_send_sems.at[o],
                recv_sem=rs_recv_sems.at[o],
                device_id=(partner,),
                device_id_type=pl.DeviceIdType.MESH,
            )
            rdma.start()
            rs.append(rdma)
        for rdma in rs:
            rdma.wait_recv()
        red = sbuf_ref[my].astype(jnp.float32) + jnp.sum(
            rsbuf_ref[1:N_DEV].astype(jnp.float32), axis=0)
        redb_ref[...] = red.astype(jnp.bfloat16)
        for rdma in rs:
            rdma.wait_send()

        my_dst = out_ref.at[my // SPB, pl.ds((my % SPB) * CH, CH), :]
        ag = []
        for o in range(1, N_DEV):
            rdma = pltpu.make_async_remote_copy(
                src_ref=redb_ref,
                dst_ref=my_dst,
                send_sem=ag_send_sems.at[o],
                recv_sem=ag_recv_sems.at[o],
                device_id=(my ^ o,),
                device_id_type=pl.DeviceIdType.MESH,
            )
            rdma.start()
            ag.append(rdma)
        out_ref[my // SPB, pl.ds((my % SPB) * CH, CH), :] = redb_ref[...]
        for rdma in ag:
            rdma.wait_recv()
        for rdma in ag:
            rdma.wait_send()

    return pl.pallas_call(
        body,
        out_shape=jax.ShapeDtypeStruct((B, Sq, Dout), jnp.bfloat16),
        in_specs=[pl.BlockSpec(memory_space=pltpu.VMEM)] * 5,
        out_specs=pl.BlockSpec(memory_space=pltpu.VMEM),
        compiler_params=pltpu.CompilerParams(collective_id=0),
        scratch_shapes=[
            pltpu.VMEM((N_DEV, CH, Dout), jnp.bfloat16),
            pltpu.VMEM((N_DEV, CH, Dout), jnp.bfloat16),
            pltpu.VMEM((CH, Dout), jnp.bfloat16),
            pltpu.SemaphoreType.DMA((N_DEV,)),
            pltpu.SemaphoreType.DMA((N_DEV,)),
            pltpu.SemaphoreType.DMA((N_DEV,)),
            pltpu.SemaphoreType.DMA((N_DEV,)),
        ],
    )(x, Wq, K_ext, V_ext, Wo)


# device time: 28702 ns/iter; 1.2912x vs baseline; 1.1946x over previous
import jax
import jax.numpy as jnp
from jax import lax
from jax.experimental import pallas as pl
from jax.experimental.pallas import tpu as pltpu

N_DEV = 16


def kernel(x, Wq, K_ext, V_ext, Wo):
    B, Sq, D = x.shape
    _, Skv, Hl, Dh = K_ext.shape
    Dq = Hl * Dh
    Dout = Wo.shape[1]
    CH = B * Sq // N_DEV
    SPB = Sq // CH

    my_idx = lax.axis_index("i")
    wq_slice = lax.dynamic_slice(Wq, (0, my_idx * Dq), (D, Dq))
    wo_slice = lax.dynamic_slice(Wo, (my_idx * Dq, 0), (Dq, Dout))

    def body(x_ref, wq_ref, k_ref, v_ref, wo_ref, out_ref,
             sbuf_ref, rsbuf_ref, redb_ref,
             rs_send_sems, rs_recv_sems, ag_send_sems, ag_recv_sems):
        my = lax.axis_index("i")

        barrier_sem = pltpu.get_barrier_semaphore()
        for o in range(1, N_DEV):
            pl.semaphore_signal(
                barrier_sem, inc=1,
                device_id=(my ^ o,),
                device_id_type=pl.DeviceIdType.MESH,
            )

        xb = x_ref[...].reshape(B * Sq, D).astype(jnp.bfloat16)
        wq = wq_ref[...].astype(jnp.bfloat16)
        q = jnp.dot(xb, wq, preferred_element_type=jnp.float32)

        qi = lax.broadcasted_iota(jnp.int32, (Sq, Skv), 0) // 64
        ki = lax.broadcasted_iota(jnp.int32, (Sq, Skv), 1) // 64
        mask = (qi == ki) | (ki == 0) | ((qi + ki) % 3 == 0)

        ctx_rows = []
        for b in range(B):
            heads = []
            for h in range(Hl):
                qh = q[b * Sq:(b + 1) * Sq, h * Dh:(h + 1) * Dh]
                kh = k_ref[b, :, h, :].astype(jnp.bfloat16)
                s = lax.dot_general(
                    qh.astype(jnp.bfloat16), kh,
                    (((1,), (1,)), ((), ())),
                    preferred_element_type=jnp.float32,
                ) * 0.125
                s = jnp.where(mask, s, -1e9)
                m = jnp.max(s, axis=1, keepdims=True)
                w = jnp.exp(s - m)
                w = w / jnp.sum(w, axis=1, keepdims=True)
                vh = v_ref[b, :, h, :].astype(jnp.bfloat16)
                heads.append(jnp.dot(w.astype(jnp.bfloat16), vh,
                                     preferred_element_type=jnp.float32))
            ctx_rows.append(jnp.concatenate(heads, axis=1))
        ctx = jnp.concatenate(ctx_rows, axis=0)

        wo = wo_ref[...].astype(jnp.bfloat16)
        sbuf_ref[...] = jnp.dot(
            ctx.astype(jnp.bfloat16), wo,
            preferred_element_type=jnp.float32,
        ).astype(jnp.bfloat16).reshape(N_DEV, CH, Dout)
        pl.semaphore_wait(barrier_sem, N_DEV - 1)

        rs = []
        for o in range(1, N_DEV):
            partner = my ^ o
            rdma = pltpu.make_async_remote_copy(
                src_ref=sbuf_ref.at[partner],
                dst_ref=rsbuf_ref.at[o],
                send_sem=rs_send_sems.at[o],
                recv_sem=rs_recv_sems.at[o],
                device_id=(partner,),
                device_id_type=pl.DeviceIdType.MESH,
            )
            rdma.start()
            rs.append(rdma)
        for rdma in rs:
            rdma.wait_recv()
        red = sbuf_ref[my].astype(jnp.float32) + jnp.sum(
            rsbuf_ref[1:N_DEV].astype(jnp.float32), axis=0)
        redb_ref[...] = red.astype(jnp.bfloat16)
        for rdma in rs:
            rdma.wait_send()

        my_dst = out_ref.at[my // SPB, pl.ds((my % SPB) * CH, CH), :]
        ag = []
        for o in range(1, N_DEV):
            rdma = pltpu.make_async_remote_copy(
                src_ref=redb_ref,
                dst_ref=my_dst,
                send_sem=ag_send_sems.at[o],
                recv_sem=ag_recv_sems.at[o],
                device_id=(my ^ o,),
                device_id_type=pl.DeviceIdType.MESH,
            )
            rdma.start()
            ag.append(rdma)
        out_ref[my // SPB, pl.ds((my % SPB) * CH, CH), :] = redb_ref[...]
        for rdma in ag:
            rdma.wait_recv()
        for rdma in ag:
            rdma.wait_send()

    return pl.pallas_call(
        body,
        out_shape=jax.ShapeDtypeStruct((B, Sq, Dout), jnp.bfloat16),
        in_specs=[pl.BlockSpec(memory_space=pltpu.VMEM)] * 5,
        out_specs=pl.BlockSpec(memory_space=pltpu.VMEM),
        compiler_params=pltpu.CompilerParams(collective_id=0),
        scratch_shapes=[
            pltpu.VMEM((N_DEV, CH, Dout), jnp.bfloat16),
            pltpu.VMEM((N_DEV, CH, Dout), jnp.bfloat16),
            pltpu.VMEM((CH, Dout), jnp.bfloat16),
            pltpu.SemaphoreType.DMA((N_DEV,)),
            pltpu.SemaphoreType.DMA((N_DEV,)),
            pltpu.SemaphoreType.DMA((N_DEV,)),
            pltpu.SemaphoreType.DMA((N_DEV,)),
        ],
    )(x, wq_slice, K_ext, V_ext, wo_slice)
